# Initial kernel scaffold; baseline (speedup 1.0000x reference)
import jax
import jax.numpy as jnp
from jax.experimental import pallas as pl


def kernel(x, segment_ids, W, b):
    raise NotImplementedError("write your pallas kernel here")


# speedup vs baseline: 2.8232x; 2.8232x over previous
import functools

import jax
import jax.numpy as jnp
from jax import lax
from jax.experimental import pallas as pl
from jax.experimental.pallas import tpu as pltpu
from jax.experimental.pallas import tpu_sc as plsc

N_NODES = 100000
D = 128
G = 512
NCLS = 10
L = 16
NC = 2
NS = 16
NW = NC * NS
CB = 640
KMAX = 5
PER_W = CB * KMAX
NV = D // L


def _sc_segment_sum(x1d, seg):
    mesh = plsc.VectorSubcoreMesh(core_axis_name="c", subcore_axis_name="s")

    @functools.partial(
        pl.kernel,
        out_type=(
            jax.ShapeDtypeStruct((NC, G * D), jnp.float32),
            jax.ShapeDtypeStruct((NW, 2 * D), jnp.float32),
            jax.ShapeDtypeStruct((NW, 2 * L), jnp.int32),
        ),
        mesh=mesh,
        scratch_types=[
            pltpu.VMEM((CB * D,), jnp.float32),
            pltpu.VMEM((CB,), jnp.int32),
            pltpu.VMEM((D,), jnp.float32),
            pltpu.VMEM((2 * D,), jnp.float32),
            pltpu.VMEM((2 * L,), jnp.int32),
            pltpu.VMEM((32 * D,), jnp.float32),
        ],
    )
    def body(x_hbm, ids_hbm, dense_hbm, bsum_hbm, bid_hbm,
             xbuf, idsbuf, stage, bbuf, bidbuf, zbuf):
        cid = lax.axis_index("c")
        sid = lax.axis_index("s")
        wid = sid * NC + cid
        zero = jnp.zeros((L,), jnp.float32)
        ones_i = jnp.full((L,), 1, jnp.int32)

        for i in range(32 * NV):
            zbuf[pl.ds(i * L, L)] = zero
        pltpu.sync_copy(zbuf, dense_hbm.at[cid, pl.ds(sid * 32 * D, 32 * D)])

        for k in range(NV):
            bbuf[pl.ds(k * L, L)] = zero
        bidbuf[pl.ds(0, L)] = ones_i * -1

        plsc.subcore_barrier()

        base = wid * PER_W
        rows_w = jnp.minimum(PER_W, N_NODES - base)
        n_chunks = (rows_w + CB - 1) // CB

        def row_step(rid, roff, carry):
            cur, nf, sums = carry[0], carry[1], carry[2:]
            changed = rid != cur

            @pl.when(changed & (nf == 0))
            def _():
                for k in range(NV):
                    bbuf[pl.ds(k * L, L)] = sums[k]
                bidbuf[pl.ds(0, L)] = ones_i * cur

            @pl.when(changed & (nf >= 1))
            def _():
                for k in range(NV):
                    stage[pl.ds(k * L, L)] = sums[k]
                pltpu.sync_copy(stage, dense_hbm.at[cid, pl.ds(cur * D, D)])

            new_sums = tuple(
                jnp.where(changed, zero, sums[k])
                + xbuf[pl.ds(roff + k * L, L)]
                for k in range(NV)
            )
            nf = nf + changed.astype(jnp.int32)
            return (rid, nf) + new_sums

        def group_body(g, carry):
            idvec = idsbuf[pl.ds(g * L, L)]
            for k in range(L):
                carry = row_step(idvec[k], (g * L + k) * D, carry)
            return carry

        def chunk_body(j, carry):
            s = base + j * CB
            win = jnp.minimum(s, N_NODES - CB)
            g_lo = lax.div(s - win, L)
            g_hi = lax.div(jnp.minimum(s + CB, N_NODES) - win, L)
            pltpu.sync_copy(x_hbm.at[pl.ds(win * D, CB * D)], xbuf)
            pltpu.sync_copy(ids_hbm.at[pl.ds(win, CB)], idsbuf)
            return lax.fori_loop(g_lo, g_hi, group_body, carry)

        init = (jnp.int32(-1), jnp.int32(-1)) + tuple(zero for _ in range(NV))
        carry = lax.fori_loop(0, n_chunks, chunk_body, init)

        cur, sums = carry[0], carry[2:]
        for k in range(NV):
            bbuf[pl.ds(D + k * L, L)] = sums[k]
        bidbuf[pl.ds(L, L)] = ones_i * cur

        pltpu.sync_copy(bbuf, bsum_hbm.at[wid])
        pltpu.sync_copy(bidbuf, bid_hbm.at[wid])

    return body(x1d, seg)


def _head(pd, bsums, bids, W, b2):
    def body(pd_ref, bs_ref, bi_ref, w_ref, b_ref, o_ref):
        y = pd_ref[0] + pd_ref[1]
        bidv = bi_ref[...][:, 0]
        oh = (lax.broadcasted_iota(jnp.int32, (G, 2 * NW), 0)
              == bidv[None, :]).astype(jnp.float32)
        y = y + jnp.dot(oh, bs_ref[...], preferred_element_type=jnp.float32)
        y = jnp.maximum(y, 0.0)
        o_ref[...] = (
            jnp.dot(y, w_ref[...], preferred_element_type=jnp.float32)
            + b_ref[...]
        )

    return pl.pallas_call(
        body,
        out_shape=jax.ShapeDtypeStruct((G, NCLS), jnp.float32),
    )(pd, bsums, bids, W, b2)


def kernel(x, segment_ids, W, b):
    seg = segment_ids.astype(jnp.int32)
    pd, bsums, bids = _sc_segment_sum(x.reshape(-1), seg)
    return _head(pd.reshape(NC, G, D), bsums.reshape(2 * NW, D),
                 bids.reshape(2 * NW, L), W, b.reshape(1, NCLS))


# speedup vs baseline: 4.8496x; 1.7178x over previous
import functools

import jax
import jax.numpy as jnp
from jax import lax
from jax.experimental import pallas as pl
from jax.experimental.pallas import tpu as pltpu
from jax.experimental.pallas import tpu_sc as plsc

N_NODES = 100000
D = 128
G = 512
NCLS = 10
L = 16
NC = 2
NS = 16
NW = NC * NS
CB = 320
KMAX = 10
PER_W = CB * KMAX
NV = D // L


def _sc_segment_sum(x1d, seg):
    mesh = plsc.VectorSubcoreMesh(core_axis_name="c", subcore_axis_name="s")

    @functools.partial(
        pl.kernel,
        out_type=(
            jax.ShapeDtypeStruct((NC, G * D), jnp.float32),
            jax.ShapeDtypeStruct((NW, 2 * D), jnp.float32),
            jax.ShapeDtypeStruct((NW, 2 * L), jnp.int32),
        ),
        mesh=mesh,
        scratch_types=[
            pltpu.VMEM((CB * D,), jnp.float32),
            pltpu.VMEM((CB * D,), jnp.float32),
            pltpu.VMEM((CB,), jnp.int32),
            pltpu.VMEM((CB,), jnp.int32),
            pltpu.VMEM((D,), jnp.float32),
            pltpu.VMEM((D,), jnp.float32),
            pltpu.VMEM((2 * D,), jnp.float32),
            pltpu.VMEM((2 * L,), jnp.int32),
            pltpu.VMEM((32 * D,), jnp.float32),
            pltpu.SemaphoreType.DMA,
            pltpu.SemaphoreType.DMA,
        ],
    )
    def body(x_hbm, ids_hbm, dense_hbm, bsum_hbm, bid_hbm,
             xbuf0, xbuf1, idsbuf0, idsbuf1, stage, sumbuf, bbuf, bidbuf,
             zbuf, sem0, sem1):
        cid = lax.axis_index("c")
        sid = lax.axis_index("s")
        wid = sid * NC + cid
        zero = jnp.zeros((L,), jnp.float32)
        ones_i = jnp.full((L,), 1, jnp.int32)
        xbufs, idsbufs, sems = (xbuf0, xbuf1), (idsbuf0, idsbuf1), (sem0, sem1)

        for i in range(32 * NV):
            zbuf[pl.ds(i * L, L)] = zero
        pltpu.sync_copy(zbuf, dense_hbm.at[cid, pl.ds(sid * 32 * D, 32 * D)])

        for k in range(NV):
            bbuf[pl.ds(k * L, L)] = zero
        bidbuf[pl.ds(0, L)] = ones_i * -1

        plsc.subcore_barrier()

        base = wid * PER_W

        def win_of(s):
            return jnp.minimum(s, N_NODES - CB)

        def start_fetch(j, buf):
            win = win_of(base + j * CB)
            pltpu.async_copy(x_hbm.at[pl.ds(win * D, CB * D)],
                             xbufs[buf], sems[buf])
            pltpu.async_copy(ids_hbm.at[pl.ds(win, CB)],
                             idsbufs[buf], sems[buf])

        def wait_fetch(buf):
            pltpu.make_async_copy(x_hbm.at[pl.ds(0, CB * D)],
                                  xbufs[buf], sems[buf]).wait()
            pltpu.make_async_copy(ids_hbm.at[pl.ds(0, CB)],
                                  idsbufs[buf], sems[buf]).wait()

        def tree_add(vals):
            vals = list(vals)
            while len(vals) > 1:
                nxt = [vals[i] + vals[i + 1] for i in range(0, len(vals) - 1, 2)]
                if len(vals) % 2:
                    nxt.append(vals[-1])
                vals = nxt
            return vals[0]

        def row_step(rid, roff, xbuf, carry):
            cur, nf, sums = carry[0], carry[1], carry[2:]
            changed = rid != cur

            @pl.when(changed & (nf == 0))
            def _():
                for k in range(NV):
                    bbuf[pl.ds(k * L, L)] = sums[k]
                bidbuf[pl.ds(0, L)] = ones_i * cur

            @pl.when(changed & (nf >= 1))
            def _():
                for k in range(NV):
                    stage[pl.ds(k * L, L)] = sums[k]
                pltpu.sync_copy(stage, dense_hbm.at[cid, pl.ds(cur * D, D)])

            new_sums = tuple(
                jnp.where(changed, zero, sums[k])
                + xbuf[pl.ds(roff + k * L, L)]
                for k in range(NV)
            )
            nf = nf + changed.astype(jnp.int32)
            return (rid, nf) + new_sums

        def make_group_body(xbuf, idsbuf):
            def group_body(g, carry):
                idvec = idsbuf[pl.ds(g * L, L)]
                gbase = g * (L * D)
                same = (idvec[0] == idvec[L - 1]) & (idvec[0] == carry[0])

                def fast(c):
                    for k in range(NV):
                        acc = tree_add(
                            [xbuf[pl.ds(gbase + r * D + k * L, L)]
                             for r in range(L)])
                        sumbuf[pl.ds(k * L, L)] = sumbuf[pl.ds(k * L, L)] + acc
                    return c

                def slow(c):
                    full = c + tuple(sumbuf[pl.ds(k * L, L)]
                                     for k in range(NV))
                    for k in range(L):
                        full = row_step(idvec[k], gbase + k * D, xbuf, full)
                    for k in range(NV):
                        sumbuf[pl.ds(k * L, L)] = full[2 + k]
                    return full[:2]

                return lax.cond(same, fast, slow, carry)
            return group_body

        def process(j, buf, carry):
            s = base + j * CB
            win = win_of(s)
            g_lo = lax.div(s - win, L)
            g_hi = lax.div(jnp.minimum(s + CB, N_NODES) - win, L)
            return lax.fori_loop(g_lo, g_hi,
                                 make_group_body(xbufs[buf], idsbufs[buf]),
                                 carry)

        carry = (jnp.int32(-1), jnp.int32(-1))
        for k in range(NV):
            sumbuf[pl.ds(k * L, L)] = zero

        start_fetch(0, 0)

        def pair_body(p, carry):
            j = p * 2
            wait_fetch(0)
            start_fetch(j + 1, 1)
            carry = process(j, 0, carry)
            wait_fetch(1)
            start_fetch(j + 2, 0)
            carry = process(j + 1, 1, carry)
            return carry

        carry = lax.fori_loop(0, KMAX // 2, pair_body, carry)
        wait_fetch(0)

        cur = carry[0]
        for k in range(NV):
            bbuf[pl.ds(D + k * L, L)] = sumbuf[pl.ds(k * L, L)]
        bidbuf[pl.ds(L, L)] = ones_i * cur

        pltpu.sync_copy(bbuf, bsum_hbm.at[wid])
        pltpu.sync_copy(bidbuf, bid_hbm.at[wid])

    return body(x1d, seg)


def _head(pd, bsums, bids, W, b2):
    def body(pd_ref, bs_ref, bi_ref, w_ref, b_ref, o_ref):
        y = pd_ref[0] + pd_ref[1]
        bidv = bi_ref[...][:, 0]
        oh = (lax.broadcasted_iota(jnp.int32, (G, 2 * NW), 0)
              == bidv[None, :]).astype(jnp.float32)
        y = y + jnp.dot(oh, bs_ref[...], preferred_element_type=jnp.float32)
        y = jnp.maximum(y, 0.0)
        o_ref[...] = (
            jnp.dot(y, w_ref[...], preferred_element_type=jnp.float32)
            + b_ref[...]
        )

    return pl.pallas_call(
        body,
        out_shape=jax.ShapeDtypeStruct((G, NCLS), jnp.float32),
    )(pd, bsums, bids, W, b2)


def kernel(x, segment_ids, W, b):
    seg = segment_ids.astype(jnp.int32)
    pd, bsums, bids = _sc_segment_sum(x.reshape(-1), seg)
    return _head(pd.reshape(NC, G, D), bsums.reshape(2 * NW, D),
                 bids.reshape(2 * NW, L), W, b.reshape(1, NCLS))
